# baseline (speedup 1.0000x reference)
import functools

import jax
import jax.numpy as jnp
from jax import lax
from jax.experimental import pallas as pl
from jax.experimental.pallas import tpu as pltpu
from jax.experimental.pallas import tpu_sc as plsc

_VOCAB = 100000
_EMBED = 300
_NSTEP = 25
_GROW = 125
_CHUNK = 9600
_RPC = 32

_NC = 2
_NS = 16
_NW = _NC * _NS
_CH = 128


def _scores_body(p_ref, w_ref, b_ref, out_ref):
    x = p_ref[0]
    w = w_ref[...]
    out_ref[0] = jnp.dot(x, w, preferred_element_type=jnp.float32) + b_ref[0]


def _compute_scores(pretrained, W, b):
    k = jnp.arange(_CHUNK)[:, None] - _EMBED * jnp.arange(_RPC)[None, :]
    w_mat = jnp.where(
        (k >= 0) & (k < _EMBED), W[0][jnp.clip(k, 0, _EMBED - 1)], 0.0
    )
    table = pretrained.reshape(_NSTEP, _GROW, _CHUNK)
    out = pl.pallas_call(
        _scores_body,
        grid=(_NSTEP,),
        in_specs=[
            pl.BlockSpec((1, _GROW, _CHUNK), lambda i: (i, 0, 0)),
            pl.BlockSpec((_CHUNK, _RPC), lambda i: (0, 0)),
            pl.BlockSpec(memory_space=pltpu.SMEM),
        ],
        out_specs=pl.BlockSpec((1, _GROW, _RPC), lambda i: (i, 0, 0)),
        out_shape=jax.ShapeDtypeStruct((_NSTEP, _GROW, _RPC), jnp.float32),
    )(table, w_mat, b)
    return out.reshape(_VOCAB)


def _make_gather(n_total):
    per_w = n_total // _NW
    nch = per_w // _CH
    mesh = plsc.VectorSubcoreMesh(core_axis_name="c", subcore_axis_name="s")

    @functools.partial(
        pl.kernel,
        mesh=mesh,
        out_type=jax.ShapeDtypeStruct((_NW, nch, _CH), jnp.float32),
        scratch_types=[
            pltpu.VMEM((nch, _CH), jnp.int32),
            pltpu.VMEM((nch, _CH), jnp.float32),
            pltpu.SemaphoreType.DMA,
        ],
    )
    def gather(scores_hbm, idx_hbm, out_hbm, idx_v, vals_v, sem):
        wid = lax.axis_index("s") * _NC + lax.axis_index("c")
        pltpu.sync_copy(idx_hbm.at[wid], idx_v)

        def fire(j, carry):
            pltpu.make_async_copy(scores_hbm.at[idx_v.at[j]], vals_v.at[j], sem).start()
            return carry

        def drain(j, carry):
            pltpu.make_async_copy(scores_hbm.at[idx_v.at[j]], vals_v.at[j], sem).wait()
            return carry

        lax.fori_loop(0, nch, fire, 0)
        lax.fori_loop(0, nch, drain, 0)
        pltpu.sync_copy(vals_v, out_hbm.at[wid])

    return gather


def kernel(input, pretrained, W, b):
    batch, hist = input.shape
    n_total = batch * hist
    scores = _compute_scores(pretrained, W, b)
    idx = input.astype(jnp.int32).reshape(_NW, n_total // (_NW * _CH), _CH)
    out = _make_gather(n_total)(scores, idx)
    return out.reshape(batch, hist)


# speedup vs baseline: 11.6507x; 11.6507x over previous
import functools

import jax
import jax.numpy as jnp
from jax import lax
from jax.experimental import pallas as pl
from jax.experimental.pallas import tpu as pltpu
from jax.experimental.pallas import tpu_sc as plsc

_VOCAB = 100000
_EMBED = 300
_RBLK = 10000

_NC = 2
_NS = 16
_NW = _NC * _NS
_CH = 128


def _scores_body(p_ref, w_ref, b_ref, out_ref):
    x = p_ref[...]
    w = w_ref[...]
    mm = jnp.dot(x, w, preferred_element_type=jnp.float32)
    out_ref[...] = mm[:, 0:1] + b_ref[0]


def _compute_scores(pretrained, W, b):
    nblk = _VOCAB // _RBLK
    w_mat = jnp.zeros((_EMBED, 128), jnp.float32).at[:, 0].set(W[0])
    out = pl.pallas_call(
        _scores_body,
        grid=(nblk,),
        in_specs=[
            pl.BlockSpec((_RBLK, _EMBED), lambda i: (i, 0)),
            pl.BlockSpec((_EMBED, 128), lambda i: (0, 0)),
            pl.BlockSpec(memory_space=pltpu.SMEM),
        ],
        out_specs=pl.BlockSpec((_RBLK, 1), lambda i: (i, 0)),
        out_shape=jax.ShapeDtypeStruct((_VOCAB, 1), jnp.float32),
    )(pretrained, w_mat, b)
    return out.reshape(_VOCAB)


def _make_gather(n_total):
    per_w = n_total // _NW
    nch = per_w // _CH
    mesh = plsc.VectorSubcoreMesh(core_axis_name="c", subcore_axis_name="s")

    @functools.partial(
        pl.kernel,
        mesh=mesh,
        out_type=jax.ShapeDtypeStruct((_NW, nch, _CH), jnp.float32),
        scratch_types=[
            pltpu.VMEM((nch, _CH), jnp.int32),
            pltpu.VMEM((nch, _CH), jnp.float32),
            pltpu.SemaphoreType.DMA,
        ],
    )
    def gather(scores_hbm, idx_hbm, out_hbm, idx_v, vals_v, sem):
        wid = lax.axis_index("s") * _NC + lax.axis_index("c")
        pltpu.sync_copy(idx_hbm.at[wid], idx_v)

        def fire(j, carry):
            pltpu.make_async_copy(scores_hbm.at[idx_v.at[j]], vals_v.at[j], sem).start()
            return carry

        def drain(j, carry):
            pltpu.make_async_copy(scores_hbm.at[idx_v.at[j]], vals_v.at[j], sem).wait()
            return carry

        lax.fori_loop(0, nch, fire, 0)
        lax.fori_loop(0, nch, drain, 0)
        pltpu.sync_copy(vals_v, out_hbm.at[wid])

    return gather


def kernel(input, pretrained, W, b):
    batch, hist = input.shape
    n_total = batch * hist
    scores = _compute_scores(pretrained, W, b)
    idx = input.astype(jnp.int32).reshape(_NW, n_total // (_NW * _CH), _CH)
    out = _make_gather(n_total)(scores, idx)
    return out.reshape(batch, hist)
